# baseline (speedup 1.0000x reference)
import functools

import jax
import jax.numpy as jnp
from jax import lax
from jax.experimental import pallas as pl
from jax.experimental.pallas import tpu as pltpu
from jax.experimental.pallas import tpu_sc as plsc

_CHUNK = 2048
_BB = 1


def _add_kernel(x_ref, p_ref, o_ref):
    o_ref[...] = x_ref[...] + p_ref[...][None, :, :]


def kernel(inputs, pos_table):
    b, s, d = inputs.shape
    chunk = min(_CHUNK, s)
    bb = min(_BB, b)
    return pl.pallas_call(
        _add_kernel,
        grid=(s // chunk, b // bb),
        in_specs=[
            pl.BlockSpec((bb, chunk, d), lambda i, j: (j, i, 0)),
            pl.BlockSpec((chunk, d), lambda i, j: (i, 0)),
        ],
        out_specs=pl.BlockSpec((bb, chunk, d), lambda i, j: (j, i, 0)),
        out_shape=jax.ShapeDtypeStruct((b, s, d), inputs.dtype),
    )(inputs, pos_table)


_tc_kernel = kernel


_NC = 2
_NS = 16
_NW = _NC * _NS
_CH_SC = 32


def _kernel_sc(inputs, pos_table):
    b, s, d = inputs.shape
    spw = s // _NW
    nv = _CH_SC * d // 16
    mesh = plsc.VectorSubcoreMesh(
        core_axis_name="c", subcore_axis_name="s", num_cores=_NC, num_subcores=_NS
    )

    @functools.partial(
        pl.kernel,
        mesh=mesh,
        out_type=jax.ShapeDtypeStruct((b * s * d,), inputs.dtype),
        scratch_types=[
            pltpu.VMEM((_CH_SC * d,), jnp.float32),
            pltpu.VMEM((_CH_SC * d,), jnp.float32),
        ],
    )
    def k(in_hbm, tab_hbm, out_hbm, t_v, x_v):
        wid = lax.axis_index("s") * _NC + lax.axis_index("c")
        s_base = wid * spw
        for c in range(spw // _CH_SC):
            s0 = s_base + c * _CH_SC
            pltpu.sync_copy(tab_hbm.at[pl.ds(s0 * d, _CH_SC * d)], t_v)
            for bi in range(b):
                r0 = (bi * s + s0) * d
                pltpu.sync_copy(in_hbm.at[pl.ds(r0, _CH_SC * d)], x_v)

                def add_body(kk, _, base=0):
                    off = kk * 16
                    plsc.addupdate(x_v.at[pl.ds(off, 16)], t_v[pl.ds(off, 16)])
                    return 0

                lax.fori_loop(0, nv, add_body, 0)
                pltpu.sync_copy(x_v, out_hbm.at[pl.ds(r0, _CH_SC * d)])

    out = k(inputs.reshape(-1), pos_table.reshape(-1))
    return out.reshape(b, s, d)


kernel = _kernel_sc


# speedup vs baseline: 1.3839x; 1.3839x over previous
import functools

import jax
import jax.numpy as jnp
from jax import lax
from jax.experimental import pallas as pl
from jax.experimental.pallas import tpu as pltpu
from jax.experimental.pallas import tpu_sc as plsc

_CHUNK = 2048
_BB = 1


def _add_kernel(x_ref, p_ref, o_ref):
    o_ref[...] = x_ref[...] + p_ref[...][None, :, :]


def kernel(inputs, pos_table):
    b, s, d = inputs.shape
    chunk = min(_CHUNK, s)
    bb = min(_BB, b)
    return pl.pallas_call(
        _add_kernel,
        grid=(s // chunk, b // bb),
        in_specs=[
            pl.BlockSpec((bb, chunk, d), lambda i, j: (j, i, 0)),
            pl.BlockSpec((chunk, d), lambda i, j: (i, 0)),
        ],
        out_specs=pl.BlockSpec((bb, chunk, d), lambda i, j: (j, i, 0)),
        out_shape=jax.ShapeDtypeStruct((b, s, d), inputs.dtype),
    )(inputs, pos_table)


_tc_kernel = kernel


_NC = 2
_NS = 16
_NW = _NC * _NS
_CH_SC = 32


def _kernel_sc(inputs, pos_table):
    b, s, d = inputs.shape
    spw = s // _NW
    nv = _CH_SC * d // 16
    mesh = plsc.VectorSubcoreMesh(
        core_axis_name="c", subcore_axis_name="s", num_cores=_NC, num_subcores=_NS
    )

    @functools.partial(
        pl.kernel,
        mesh=mesh,
        out_type=jax.ShapeDtypeStruct((b * s * d,), inputs.dtype),
        scratch_types=[
            pltpu.VMEM((_CH_SC * d,), jnp.float32),
            pltpu.VMEM((_CH_SC * d,), jnp.float32),
        ],
    )
    def k(in_hbm, tab_hbm, out_hbm, t_v, x_v):
        wid = lax.axis_index("s") * _NC + lax.axis_index("c")
        s_base = wid * spw
        for c in range(spw // _CH_SC):
            s0 = s_base + c * _CH_SC
            pltpu.sync_copy(tab_hbm.at[pl.ds(s0 * d, _CH_SC * d)], t_v)
            for bi in range(b):
                r0 = (bi * s + s0) * d
                pltpu.sync_copy(in_hbm.at[pl.ds(r0, _CH_SC * d)], x_v)

                @plsc.parallel_loop(0, nv * 16, 16, unroll=8)
                def add_body(off):
                    plsc.addupdate(x_v.at[pl.ds(off, 16)], t_v[pl.ds(off, 16)])

                pltpu.sync_copy(x_v, out_hbm.at[pl.ds(r0, _CH_SC * d)])

    out = k(inputs.reshape(-1), pos_table.reshape(-1))
    return out.reshape(b, s, d)


kernel = _kernel_sc


# speedup vs baseline: 7.1738x; 5.1838x over previous
import jax
import jax.numpy as jnp
from jax.experimental import pallas as pl

_CHUNK = 2048
_BB = 1


def _add_kernel(x_ref, p_ref, o_ref):
    o_ref[...] = x_ref[...] + p_ref[...][None, :, :]


def kernel(inputs, pos_table):
    b, s, d = inputs.shape
    chunk = min(_CHUNK, s)
    bb = min(_BB, b)
    return pl.pallas_call(
        _add_kernel,
        grid=(s // chunk, b // bb),
        in_specs=[
            pl.BlockSpec((bb, chunk, d), lambda i, j: (j, i, 0)),
            pl.BlockSpec((chunk, d), lambda i, j: (i, 0)),
        ],
        out_specs=pl.BlockSpec((bb, chunk, d), lambda i, j: (j, i, 0)),
        out_shape=jax.ShapeDtypeStruct((b, s, d), inputs.dtype),
    )(inputs, pos_table)
